# baseline (speedup 1.0000x reference)
import functools

import jax
import jax.numpy as jnp
from jax import lax
from jax.experimental import pallas as pl
from jax.experimental.pallas import tpu as pltpu
from jax.experimental.pallas import tpu_sc as plsc

D_MODEL = 4096

_NC = 2
_NS = 16
_NW = _NC * _NS
_CHUNK = 8


def _sc_gather(table, ids3):
    n_ch = ids3.shape[1]
    rows_per_w = n_ch * _CHUNK
    total = _NW * rows_per_w
    d = table.shape[1]
    mesh = plsc.VectorSubcoreMesh(core_axis_name="c", subcore_axis_name="s")

    @functools.partial(
        pl.kernel,
        mesh=mesh,
        out_type=jax.ShapeDtypeStruct((total, d), jnp.float32),
        scratch_types=[
            pltpu.VMEM((n_ch, _CHUNK), jnp.int32),
            pltpu.VMEM((2, _CHUNK, d), jnp.float32),
            pltpu.SemaphoreType.DMA,
            pltpu.SemaphoreType.DMA,
        ],
    )
    def gather_kernel(table_hbm, ids_hbm, out_hbm, idx_v, rows_v, sem_in, sem_out):
        wid = lax.axis_index("s") * _NC + lax.axis_index("c")
        base = wid * rows_per_w
        pltpu.sync_copy(ids_hbm.at[wid], idx_v)
        pltpu.make_async_copy(table_hbm.at[idx_v.at[0]], rows_v.at[0], sem_in).start()

        def body(c, _):
            par = lax.rem(c, 2)
            nxt = lax.rem(c + 1, 2)
            pltpu.make_async_copy(
                table_hbm.at[idx_v.at[c]], rows_v.at[par], sem_in).wait()

            @pl.when(c >= 1)
            def _drain_prev():
                pltpu.make_async_copy(
                    rows_v.at[nxt],
                    out_hbm.at[pl.ds(base + (c - 1) * _CHUNK, _CHUNK)],
                    sem_out).wait()

            @pl.when(c + 1 < n_ch)
            def _fetch_next():
                pltpu.make_async_copy(
                    table_hbm.at[idx_v.at[c + 1]], rows_v.at[nxt], sem_in).start()

            pltpu.make_async_copy(
                rows_v.at[par],
                out_hbm.at[pl.ds(base + c * _CHUNK, _CHUNK)],
                sem_out).start()
            return _

        lax.fori_loop(0, n_ch, body, None)
        pltpu.make_async_copy(
            rows_v.at[lax.rem(n_ch - 1, 2)],
            out_hbm.at[pl.ds(base + (n_ch - 1) * _CHUNK, _CHUNK)],
            sem_out).wait()

    return gather_kernel(table, ids3)


def _tc_linear_slab(x, w_bf16, b2, s, batch, out_prev):
    n, d = x.shape
    bt = 256
    n_t = n // bt

    def body(*refs):
        x_ref, w_ref, b_ref = refs[0], refs[1], refs[2]
        o_ref = refs[-1]
        acc = lax.dot_general(
            x_ref[...].astype(jnp.bfloat16), w_ref[...], (((1,), (1,)), ((), ())),
            preferred_element_type=jnp.float32,
        )
        o_ref[0] = acc + b_ref[...]

    in_specs = [
        pl.BlockSpec((bt, d), lambda t: (t, 0)),
        pl.BlockSpec((d, d), lambda t: (0, 0)),
        pl.BlockSpec((1, d), lambda t: (0, 0)),
    ]
    args = [x, w_bf16, b2]
    aliases = {}
    if out_prev is not None:
        in_specs.append(pl.BlockSpec(memory_space=pl.ANY))
        args.append(out_prev)
        aliases = {3: 0}
    return pl.pallas_call(
        body,
        grid=(n_t,),
        in_specs=in_specs,
        out_specs=pl.BlockSpec((1, bt, d), lambda t: (s, t, 0)),
        out_shape=jax.ShapeDtypeStruct((batch, n, d), jnp.float32),
        input_output_aliases=aliases,
        compiler_params=pltpu.CompilerParams(vmem_limit_bytes=110 * 1024 * 1024),
    )(*args)


def kernel(input_ids, embed_table, W, b):
    batch, seq = input_ids.shape
    w_bf = W.astype(jnp.bfloat16)
    b2 = b.reshape(1, -1)
    ids = input_ids.astype(jnp.int32)
    xs = []
    for s in range(batch):
        ids3 = ids[s].reshape(_NW, seq // (_NW * _CHUNK), _CHUNK)
        assert seq % (_NW * _CHUNK) == 0
        xs.append(_sc_gather(embed_table, ids3))
    out = None
    for s in range(batch):
        out = _tc_linear_slab(xs[s], w_bf, b2, s, batch, out)
    return out


# speedup vs baseline: 1.0501x; 1.0501x over previous
import functools

import jax
import jax.numpy as jnp
from jax import lax
from jax.experimental import pallas as pl
from jax.experimental.pallas import tpu as pltpu
from jax.experimental.pallas import tpu_sc as plsc

D_MODEL = 4096

_NC = 2
_NS = 16
_NW = _NC * _NS
_CHUNK = 8


def _sc_gather(table, ids3):
    n_ch = ids3.shape[1]
    rows_per_w = n_ch * _CHUNK
    total = _NW * rows_per_w
    d = table.shape[1]
    mesh = plsc.VectorSubcoreMesh(core_axis_name="c", subcore_axis_name="s")

    @functools.partial(
        pl.kernel,
        mesh=mesh,
        out_type=jax.ShapeDtypeStruct((total, d), jnp.float32),
        scratch_types=[
            pltpu.VMEM((n_ch, _CHUNK), jnp.int32),
            pltpu.VMEM((2, _CHUNK, d), jnp.float32),
            pltpu.SemaphoreType.DMA,
            pltpu.SemaphoreType.DMA,
        ],
    )
    def gather_kernel(table_hbm, ids_hbm, out_hbm, idx_v, rows_v, sem_in, sem_out):
        wid = lax.axis_index("s") * _NC + lax.axis_index("c")
        base = wid * rows_per_w
        pltpu.sync_copy(ids_hbm.at[wid], idx_v)
        pltpu.make_async_copy(table_hbm.at[idx_v.at[0]], rows_v.at[0], sem_in).start()

        def body(c, _):
            par = lax.rem(c, 2)
            nxt = lax.rem(c + 1, 2)
            pltpu.make_async_copy(
                table_hbm.at[idx_v.at[c]], rows_v.at[par], sem_in).wait()

            @pl.when(c >= 1)
            def _drain_prev():
                pltpu.make_async_copy(
                    rows_v.at[nxt],
                    out_hbm.at[pl.ds(base + (c - 1) * _CHUNK, _CHUNK)],
                    sem_out).wait()

            @pl.when(c + 1 < n_ch)
            def _fetch_next():
                pltpu.make_async_copy(
                    table_hbm.at[idx_v.at[c + 1]], rows_v.at[nxt], sem_in).start()

            pltpu.make_async_copy(
                rows_v.at[par],
                out_hbm.at[pl.ds(base + c * _CHUNK, _CHUNK)],
                sem_out).start()
            return _

        lax.fori_loop(0, n_ch, body, None)
        pltpu.make_async_copy(
            rows_v.at[lax.rem(n_ch - 1, 2)],
            out_hbm.at[pl.ds(base + (n_ch - 1) * _CHUNK, _CHUNK)],
            sem_out).wait()

    return gather_kernel(table, ids3)


_BT = 256


def _tc_linear_slab(x, w_bf16, b2, row0, n_total, out_prev):
    n, d = x.shape
    n_t = n // _BT
    t0 = row0 // _BT

    def body(*refs):
        x_ref, w_ref, b_ref = refs[0], refs[1], refs[2]
        o_ref = refs[-1]
        acc = lax.dot_general(
            x_ref[...].astype(jnp.bfloat16), w_ref[...], (((1,), (1,)), ((), ())),
            preferred_element_type=jnp.float32,
        )
        o_ref[...] = acc + b_ref[...]

    in_specs = [
        pl.BlockSpec((_BT, d), lambda t: (t, 0)),
        pl.BlockSpec((d, d), lambda t: (0, 0)),
        pl.BlockSpec((1, d), lambda t: (0, 0)),
    ]
    args = [x, w_bf16, b2]
    aliases = {}
    if out_prev is not None:
        in_specs.append(pl.BlockSpec(memory_space=pl.ANY))
        args.append(out_prev)
        aliases = {3: 0}
    return pl.pallas_call(
        body,
        grid=(n_t,),
        in_specs=in_specs,
        out_specs=pl.BlockSpec((_BT, d), lambda t: (t0 + t, 0)),
        out_shape=jax.ShapeDtypeStruct((n_total, d), jnp.float32),
        input_output_aliases=aliases,
        compiler_params=pltpu.CompilerParams(vmem_limit_bytes=110 * 1024 * 1024),
    )(*args)


_SLAB_SIZES = (4096, 12288)


def kernel(input_ids, embed_table, W, b):
    batch, seq = input_ids.shape
    n = batch * seq
    w_bf = W.astype(jnp.bfloat16)
    b2 = b.reshape(1, -1)
    ids = input_ids.reshape(-1).astype(jnp.int32)
    xs, off = [], 0
    for sz in _SLAB_SIZES:
        ids3 = lax.slice(ids, (off,), (off + sz,)).reshape(
            _NW, sz // (_NW * _CHUNK), _CHUNK)
        xs.append(_sc_gather(embed_table, ids3))
        off += sz
    out, off = None, 0
    for x_s in xs:
        out = _tc_linear_slab(x_s, w_bf, b2, off, n, out)
        off += x_s.shape[0]
    return out.reshape(batch, seq, D_MODEL)


# speedup vs baseline: 1.0513x; 1.0012x over previous
import functools

import jax
import jax.numpy as jnp
from jax import lax
from jax.experimental import pallas as pl
from jax.experimental.pallas import tpu as pltpu
from jax.experimental.pallas import tpu_sc as plsc

D_MODEL = 4096

_NC = 2
_NS = 16
_NW = _NC * _NS
_CHUNK = 8


def _sc_gather(table, ids3):
    n_ch = ids3.shape[1]
    rows_per_w = n_ch * _CHUNK
    total = _NW * rows_per_w
    d = table.shape[1]
    mesh = plsc.VectorSubcoreMesh(core_axis_name="c", subcore_axis_name="s")

    @functools.partial(
        pl.kernel,
        mesh=mesh,
        out_type=jax.ShapeDtypeStruct((total, d), jnp.float32),
        scratch_types=[
            pltpu.VMEM((n_ch, _CHUNK), jnp.int32),
            pltpu.VMEM((2, _CHUNK, d), jnp.float32),
            pltpu.SemaphoreType.DMA,
            pltpu.SemaphoreType.DMA,
        ],
    )
    def gather_kernel(table_hbm, ids_hbm, out_hbm, idx_v, rows_v, sem_in, sem_out):
        wid = lax.axis_index("s") * _NC + lax.axis_index("c")
        base = wid * rows_per_w
        pltpu.sync_copy(ids_hbm.at[wid], idx_v)
        pltpu.make_async_copy(table_hbm.at[idx_v.at[0]], rows_v.at[0], sem_in).start()

        def body(c, _):
            par = lax.rem(c, 2)
            nxt = lax.rem(c + 1, 2)
            pltpu.make_async_copy(
                table_hbm.at[idx_v.at[c]], rows_v.at[par], sem_in).wait()

            @pl.when(c >= 1)
            def _drain_prev():
                pltpu.make_async_copy(
                    rows_v.at[nxt],
                    out_hbm.at[pl.ds(base + (c - 1) * _CHUNK, _CHUNK)],
                    sem_out).wait()

            @pl.when(c + 1 < n_ch)
            def _fetch_next():
                pltpu.make_async_copy(
                    table_hbm.at[idx_v.at[c + 1]], rows_v.at[nxt], sem_in).start()

            pltpu.make_async_copy(
                rows_v.at[par],
                out_hbm.at[pl.ds(base + c * _CHUNK, _CHUNK)],
                sem_out).start()
            return _

        lax.fori_loop(0, n_ch, body, None)
        pltpu.make_async_copy(
            rows_v.at[lax.rem(n_ch - 1, 2)],
            out_hbm.at[pl.ds(base + (n_ch - 1) * _CHUNK, _CHUNK)],
            sem_out).wait()

    return gather_kernel(table, ids3)


_BT = 256


def _tc_linear_slab(x, w_bf16, b2, row0, n_total, out_prev):
    n, d = x.shape
    n_t = n // _BT
    t0 = row0 // _BT

    def body(*refs):
        x_ref, w_ref, b_ref = refs[0], refs[1], refs[2]
        o_ref = refs[-1]
        acc = lax.dot_general(
            x_ref[...].astype(jnp.bfloat16), w_ref[...], (((1,), (1,)), ((), ())),
            preferred_element_type=jnp.float32,
        )
        o_ref[...] = acc + b_ref[...]

    in_specs = [
        pl.BlockSpec((_BT, d), lambda t: (t, 0)),
        pl.BlockSpec((d, d), lambda t: (0, 0)),
        pl.BlockSpec((1, d), lambda t: (0, 0)),
    ]
    args = [x, w_bf16, b2]
    aliases = {}
    if out_prev is not None:
        in_specs.append(pl.BlockSpec(memory_space=pl.ANY))
        args.append(out_prev)
        aliases = {3: 0}
    return pl.pallas_call(
        body,
        grid=(n_t,),
        in_specs=in_specs,
        out_specs=pl.BlockSpec((_BT, d), lambda t: (t0 + t, 0)),
        out_shape=jax.ShapeDtypeStruct((n_total, d), jnp.float32),
        input_output_aliases=aliases,
        compiler_params=pltpu.CompilerParams(vmem_limit_bytes=110 * 1024 * 1024),
    )(*args)


_SLAB_SIZES = (6144, 10240)


def kernel(input_ids, embed_table, W, b):
    batch, seq = input_ids.shape
    n = batch * seq
    w_bf = W.astype(jnp.bfloat16)
    b2 = b.reshape(1, -1)
    ids = input_ids.reshape(-1).astype(jnp.int32)
    xs, off = [], 0
    for sz in _SLAB_SIZES:
        ids3 = lax.slice(ids, (off,), (off + sz,)).reshape(
            _NW, sz // (_NW * _CHUNK), _CHUNK)
        xs.append(_sc_gather(embed_table, ids3))
        off += sz
    out, off = None, 0
    for x_s in xs:
        out = _tc_linear_slab(x_s, w_bf, b2, off, n, out)
        off += x_s.shape[0]
    return out.reshape(batch, seq, D_MODEL)


# speedup vs baseline: 1.0514x; 1.0001x over previous
import functools

import jax
import jax.numpy as jnp
from jax import lax
from jax.experimental import pallas as pl
from jax.experimental.pallas import tpu as pltpu
from jax.experimental.pallas import tpu_sc as plsc

D_MODEL = 4096

_NC = 2
_NS = 16
_NW = _NC * _NS
_CHUNK = 8


def _sc_gather(table, ids3):
    n_ch = ids3.shape[1]
    rows_per_w = n_ch * _CHUNK
    total = _NW * rows_per_w
    d = table.shape[1]
    mesh = plsc.VectorSubcoreMesh(core_axis_name="c", subcore_axis_name="s")

    @functools.partial(
        pl.kernel,
        mesh=mesh,
        out_type=jax.ShapeDtypeStruct((total, d), jnp.float32),
        scratch_types=[
            pltpu.VMEM((n_ch, _CHUNK), jnp.int32),
            pltpu.VMEM((2, _CHUNK, d), jnp.float32),
            pltpu.SemaphoreType.DMA,
            pltpu.SemaphoreType.DMA,
        ],
    )
    def gather_kernel(table_hbm, ids_hbm, out_hbm, idx_v, rows_v, sem_in, sem_out):
        wid = lax.axis_index("s") * _NC + lax.axis_index("c")
        base = wid * rows_per_w
        pltpu.sync_copy(ids_hbm.at[wid], idx_v)
        pltpu.make_async_copy(table_hbm.at[idx_v.at[0]], rows_v.at[0], sem_in).start()

        def body(c, _):
            par = lax.rem(c, 2)
            nxt = lax.rem(c + 1, 2)
            pltpu.make_async_copy(
                table_hbm.at[idx_v.at[c]], rows_v.at[par], sem_in).wait()

            @pl.when(c >= 1)
            def _drain_prev():
                pltpu.make_async_copy(
                    rows_v.at[nxt],
                    out_hbm.at[pl.ds(base + (c - 1) * _CHUNK, _CHUNK)],
                    sem_out).wait()

            @pl.when(c + 1 < n_ch)
            def _fetch_next():
                pltpu.make_async_copy(
                    table_hbm.at[idx_v.at[c + 1]], rows_v.at[nxt], sem_in).start()

            pltpu.make_async_copy(
                rows_v.at[par],
                out_hbm.at[pl.ds(base + c * _CHUNK, _CHUNK)],
                sem_out).start()
            return _

        lax.fori_loop(0, n_ch, body, None)
        pltpu.make_async_copy(
            rows_v.at[lax.rem(n_ch - 1, 2)],
            out_hbm.at[pl.ds(base + (n_ch - 1) * _CHUNK, _CHUNK)],
            sem_out).wait()

    return gather_kernel(table, ids3)


_BT = 256


def _tc_linear_slab(x, w_bf16, b2, row0, n_total, out_prev):
    n, d = x.shape
    n_t = n // _BT
    t0 = row0 // _BT

    def body(*refs):
        x_ref, w_ref, b_ref = refs[0], refs[1], refs[2]
        o_ref = refs[-1]
        acc = lax.dot_general(
            x_ref[...].astype(jnp.bfloat16), w_ref[...], (((1,), (1,)), ((), ())),
            preferred_element_type=jnp.float32,
        )
        o_ref[...] = acc + b_ref[...]

    in_specs = [
        pl.BlockSpec((_BT, d), lambda t: (t, 0)),
        pl.BlockSpec((d, d), lambda t: (0, 0)),
        pl.BlockSpec((1, d), lambda t: (0, 0)),
    ]
    args = [x, w_bf16, b2]
    aliases = {}
    if out_prev is not None:
        in_specs.append(pl.BlockSpec(memory_space=pl.ANY))
        args.append(out_prev)
        aliases = {3: 0}
    return pl.pallas_call(
        body,
        grid=(n_t,),
        in_specs=in_specs,
        out_specs=pl.BlockSpec((_BT, d), lambda t: (t0 + t, 0)),
        out_shape=jax.ShapeDtypeStruct((n_total, d), jnp.float32),
        input_output_aliases=aliases,
        compiler_params=pltpu.CompilerParams(vmem_limit_bytes=110 * 1024 * 1024),
    )(*args)


def _convert_w(w):
    d = w.shape[0]
    blk = 512

    def body(w_ref, o_ref):
        o_ref[...] = w_ref[...].astype(jnp.bfloat16)

    return pl.pallas_call(
        body,
        grid=(d // blk,),
        in_specs=[pl.BlockSpec((blk, d), lambda i: (i, 0))],
        out_specs=pl.BlockSpec((blk, d), lambda i: (i, 0)),
        out_shape=jax.ShapeDtypeStruct((d, d), jnp.bfloat16),
    )(w)


_SLAB_SIZES = (6144, 10240)


def kernel(input_ids, embed_table, W, b):
    batch, seq = input_ids.shape
    n = batch * seq
    w_bf = _convert_w(W)
    b2 = b.reshape(1, -1)
    ids = input_ids.reshape(-1).astype(jnp.int32)
    xs, off = [], 0
    for sz in _SLAB_SIZES:
        ids3 = lax.slice(ids, (off,), (off + sz,)).reshape(
            _NW, sz // (_NW * _CHUNK), _CHUNK)
        xs.append(_sc_gather(embed_table, ids3))
        off += sz
    out, off = None, 0
    for x_s in xs:
        out = _tc_linear_slab(x_s, w_bf, b2, off, n, out)
        off += x_s.shape[0]
    return out.reshape(batch, seq, D_MODEL)


# speedup vs baseline: 1.0529x; 1.0014x over previous
import functools

import jax
import jax.numpy as jnp
from jax import lax
from jax.experimental import pallas as pl
from jax.experimental.pallas import tpu as pltpu
from jax.experimental.pallas import tpu_sc as plsc

D_MODEL = 4096

_NC = 2
_NS = 16
_NW = _NC * _NS
_CHUNK = 8


def _sc_gather(table, ids3):
    n_ch = ids3.shape[1]
    rows_per_w = n_ch * _CHUNK
    total = _NW * rows_per_w
    d = table.shape[1]
    mesh = plsc.VectorSubcoreMesh(core_axis_name="c", subcore_axis_name="s")

    @functools.partial(
        pl.kernel,
        mesh=mesh,
        out_type=jax.ShapeDtypeStruct((total, d), jnp.float32),
        scratch_types=[
            pltpu.VMEM((n_ch, _CHUNK), jnp.int32),
            pltpu.VMEM((2, _CHUNK, d), jnp.float32),
            pltpu.SemaphoreType.DMA,
            pltpu.SemaphoreType.DMA,
        ],
    )
    def gather_kernel(table_hbm, ids_hbm, out_hbm, idx_v, rows_v, sem_in, sem_out):
        wid = lax.axis_index("s") * _NC + lax.axis_index("c")
        base = wid * rows_per_w
        pltpu.sync_copy(ids_hbm.at[wid], idx_v)
        pltpu.make_async_copy(table_hbm.at[idx_v.at[0]], rows_v.at[0], sem_in).start()

        def body(c, _):
            par = lax.rem(c, 2)
            nxt = lax.rem(c + 1, 2)
            pltpu.make_async_copy(
                table_hbm.at[idx_v.at[c]], rows_v.at[par], sem_in).wait()

            @pl.when(c >= 1)
            def _drain_prev():
                pltpu.make_async_copy(
                    rows_v.at[nxt],
                    out_hbm.at[pl.ds(base + (c - 1) * _CHUNK, _CHUNK)],
                    sem_out).wait()

            @pl.when(c + 1 < n_ch)
            def _fetch_next():
                pltpu.make_async_copy(
                    table_hbm.at[idx_v.at[c + 1]], rows_v.at[nxt], sem_in).start()

            pltpu.make_async_copy(
                rows_v.at[par],
                out_hbm.at[pl.ds(base + c * _CHUNK, _CHUNK)],
                sem_out).start()
            return _

        lax.fori_loop(0, n_ch, body, None)
        pltpu.make_async_copy(
            rows_v.at[lax.rem(n_ch - 1, 2)],
            out_hbm.at[pl.ds(base + (n_ch - 1) * _CHUNK, _CHUNK)],
            sem_out).wait()

    return gather_kernel(table, ids3)


_BT = 256


def _tc_linear_slab(x, w_bf16, b2, row0, n_total, out_prev):
    n, d = x.shape
    n_t = n // _BT
    t0 = row0 // _BT

    def body(*refs):
        x_ref, w_ref, b_ref = refs[0], refs[1], refs[2]
        o_ref = refs[-1]
        acc = lax.dot_general(
            x_ref[...].astype(jnp.bfloat16), w_ref[...], (((1,), (1,)), ((), ())),
            preferred_element_type=jnp.float32,
        )
        o_ref[...] = acc + b_ref[...]

    in_specs = [
        pl.BlockSpec((_BT, d), lambda t: (t, 0)),
        pl.BlockSpec((d, d), lambda t: (0, 0)),
        pl.BlockSpec((1, d), lambda t: (0, 0)),
    ]
    args = [x, w_bf16, b2]
    aliases = {}
    if out_prev is not None:
        in_specs.append(pl.BlockSpec(memory_space=pl.ANY))
        args.append(out_prev)
        aliases = {3: 0}
    return pl.pallas_call(
        body,
        grid=(n_t,),
        in_specs=in_specs,
        out_specs=pl.BlockSpec((_BT, d), lambda t: (t0 + t, 0)),
        out_shape=jax.ShapeDtypeStruct((n_total, d), jnp.float32),
        input_output_aliases=aliases,
        compiler_params=pltpu.CompilerParams(vmem_limit_bytes=110 * 1024 * 1024),
    )(*args)


def _convert_w(w):
    d = w.shape[0]
    blk = 512

    def body(w_ref, o_ref):
        o_ref[...] = w_ref[...].astype(jnp.bfloat16)

    return pl.pallas_call(
        body,
        grid=(d // blk,),
        in_specs=[pl.BlockSpec((blk, d), lambda i: (i, 0))],
        out_specs=pl.BlockSpec((blk, d), lambda i: (i, 0)),
        out_shape=jax.ShapeDtypeStruct((d, d), jnp.bfloat16),
    )(w)


_SLAB_SIZES = (6144, 10240)


def kernel(input_ids, embed_table, W, b):
    batch, seq = input_ids.shape
    n = batch * seq
    w_bf = _convert_w(W)
    b2 = b.reshape(1, -1)
    ids = input_ids.reshape(-1).astype(jnp.int32)
    slabs = _SLAB_SIZES if sum(_SLAB_SIZES) == n else (n,)
    xs, off = [], 0
    for sz in slabs:
        ids3 = lax.slice(ids, (off,), (off + sz,)).reshape(
            _NW, sz // (_NW * _CHUNK), _CHUNK)
        xs.append(_sc_gather(embed_table, ids3))
        off += sz
    out, off = None, 0
    for x_s in xs:
        out = _tc_linear_slab(x_s, w_bf, b2, off, n, out)
        off += x_s.shape[0]
    return out.reshape(batch, seq, D_MODEL)
